# Initial kernel scaffold; baseline (speedup 1.0000x reference)
import jax
import jax.numpy as jnp
from jax.experimental import pallas as pl


def kernel(cll_x, mol_x, mol_edge_attr, bio_x, params, cll_edge_index, mol_edge_index, mol_batch, bio_edge_index):
    raise NotImplementedError("write your pallas kernel here")


# speedup vs baseline: 12.1531x; 12.1531x over previous
import functools

import jax
import jax.numpy as jnp
from jax import lax
from jax.experimental import pallas as pl
from jax.experimental.pallas import tpu as pltpu
from jax.experimental.pallas import tpu_sc as plsc

F32 = jnp.float32
I32 = jnp.int32

NC, NS, LN = 2, 16, 16
NW = NC * NS
CH = 128

N_CLL, E_CLL = 3451, 55216
N_MOL, E_MOL = 10000, 320000
N_BIO, E_BIO = 10000, 640000
NP_CLL = 4096
NP_B = 10240
EP_CLL = 57344
EP_MOL = 323584
EP_BIO = 655360
LAST = N_BIO - 1

_mesh = plsc.VectorSubcoreMesh(core_axis_name="c", subcore_axis_name="s")


def _relu(x):
    return jnp.maximum(x, 0.0)


def _zero_vmem2d(ref, rows, d):
    zv = jnp.zeros((LN,), F32)

    def body(i, _):
        for cc in range(d // LN):
            ref[i, pl.ds(cc * LN, LN)] = zv
        return 0

    lax.fori_loop(0, rows, body, 0)


@functools.partial(jax.jit, static_argnames=("n_out", "ep"))
def _sc_segsum(tables, src, dst, *, n_out, ep):
    d = 128
    nt = len(tables)
    npt = ep // NW
    nch = npt // CH
    rows_ps = n_out // NS

    @functools.partial(
        pl.kernel,
        out_type=jax.ShapeDtypeStruct((nt, NC, n_out, d), F32),
        mesh=_mesh,
        scratch_types=[
            pltpu.VMEM((CH,), I32),
            pltpu.VMEM((CH,), I32),
            pltpu.VMEM((CH, d), F32),
            pltpu.VMEM((64, d), F32),
            pltpu.VMEM_SHARED((n_out, d), F32),
        ],
    )
    def k(*refs):
        table_hs = refs[:nt]
        src_h, dst_h, out_h, sidx, didx, rows, zbuf, acc = refs[nt:]
        c = lax.axis_index("c")
        s = lax.axis_index("s")
        _zero_vmem2d(zbuf, 64, d)
        ebase = (c * NS + s) * npt
        for t in range(nt):
            for q in range(rows_ps // 64):
                pltpu.sync_copy(zbuf, acc.at[pl.ds(s * rows_ps + q * 64, 64)])
            plsc.subcore_barrier()

            def ch(j, _, t=t):
                b = ebase + j * CH
                pltpu.sync_copy(src_h.at[pl.ds(b, CH)], sidx)
                pltpu.sync_copy(dst_h.at[pl.ds(b, CH)], didx)
                pltpu.sync_copy(table_hs[t].at[sidx], rows)
                pltpu.sync_copy(rows, acc.at[didx], add=True)
                return 0

            lax.fori_loop(0, nch, ch, 0)
            plsc.subcore_barrier()
            pltpu.sync_copy(acc.at[pl.ds(s * rows_ps, rows_ps)],
                            out_h.at[t, c, pl.ds(s * rows_ps, rows_ps)])
            plsc.subcore_barrier()

    return k(*tables, src, dst)


@jax.jit
def _sc_degrees(cdst, bsrc, bdst):
    nch_c = EP_CLL // NW // CH
    nch_b = EP_BIO // NW // CH
    rc = NP_CLL // NS
    rb = NP_B // NS

    @functools.partial(
        pl.kernel,
        out_type=(jax.ShapeDtypeStruct((NC * NP_CLL,), F32),
                  jax.ShapeDtypeStruct((NC * NP_B,), F32),
                  jax.ShapeDtypeStruct((NC * NP_B,), F32)),
        mesh=_mesh,
        scratch_types=[
            pltpu.VMEM((CH,), I32),
            pltpu.VMEM((CH,), I32),
            pltpu.VMEM((CH,), F32),
            pltpu.VMEM((CH,), F32),
            pltpu.VMEM((rb,), F32),
            pltpu.VMEM_SHARED((NP_CLL,), F32),
            pltpu.VMEM_SHARED((NP_B,), F32),
            pltpu.VMEM_SHARED((NP_B,), F32),
        ],
    )
    def k(cdst_h, bsrc_h, bdst_h, degc_h, degb_h, flg_h,
          sidx, didx, ones, val, zb, accc, accb, accf):
        c = lax.axis_index("c")
        s = lax.axis_index("s")
        for kk in range(CH // LN):
            ones[pl.ds(kk * LN, LN)] = jnp.ones((LN,), F32)

        def zbody(i, _):
            zb[pl.ds(i * LN, LN)] = jnp.zeros((LN,), F32)
            return 0

        lax.fori_loop(0, rb // LN, zbody, 0)
        pltpu.sync_copy(zb.at[pl.ds(0, rc)], accc.at[pl.ds(s * rc, rc)])
        pltpu.sync_copy(zb, accb.at[pl.ds(s * rb, rb)])
        pltpu.sync_copy(zb, accf.at[pl.ds(s * rb, rb)])
        plsc.subcore_barrier()

        cbase = (c * NS + s) * (EP_CLL // NW)

        def chc(j, _):
            pltpu.sync_copy(cdst_h.at[pl.ds(cbase + j * CH, CH)], didx)
            pltpu.sync_copy(ones, accc.at[didx], add=True)
            return 0

        lax.fori_loop(0, nch_c, chc, 0)

        bbase = (c * NS + s) * (EP_BIO // NW)

        def chb(j, _):
            b = bbase + j * CH
            pltpu.sync_copy(bsrc_h.at[pl.ds(b, CH)], sidx)
            pltpu.sync_copy(bdst_h.at[pl.ds(b, CH)], didx)
            pltpu.sync_copy(ones, accb.at[didx], add=True)
            for kk in range(CH // LN):
                dv = didx[pl.ds(kk * LN, LN)]
                val[pl.ds(kk * LN, LN)] = jnp.where(
                    dv == LAST, jnp.ones((LN,), F32), jnp.zeros((LN,), F32))
            pltpu.sync_copy(val, accf.at[sidx], add=True)
            return 0

        lax.fori_loop(0, nch_b, chb, 0)
        plsc.subcore_barrier()
        pltpu.sync_copy(accc.at[pl.ds(s * rc, rc)],
                        degc_h.at[pl.ds(c * NP_CLL + s * rc, rc)])
        pltpu.sync_copy(accb.at[pl.ds(s * rb, rb)],
                        degb_h.at[pl.ds(c * NP_B + s * rb, rb)])
        pltpu.sync_copy(accf.at[pl.ds(s * rb, rb)],
                        flg_h.at[pl.ds(c * NP_B + s * rb, rb)])

    return k(cdst, bsrc, bdst)


def _split2(h):
    return h[:, :128], h[:, 128:256]


def _tc_y1c(xc, w, dinv):
    def body(x_r, w_r, dv_r, oa_r, ob_r):
        y = dv_r[...] * jnp.dot(x_r[...], w_r[...], preferred_element_type=F32)
        oa_r[...], ob_r[...] = _split2(y)

    return pl.pallas_call(
        body, out_shape=[jax.ShapeDtypeStruct((NP_CLL, 128), F32),
                         jax.ShapeDtypeStruct((NP_CLL, 128), F32)],
    )(xc, w, dinv)


def _tc_cll_step(agg_p, ya, yb, dinv, b, w):
    nout = w.shape[1]

    def body(aa0_r, aa1_r, ab0_r, ab1_r, ya_r, yb_r, dv_r, b_r, w_r, *outs):
        agg = jnp.concatenate([aa0_r[...] + aa1_r[...],
                               ab0_r[...] + ab1_r[...]], axis=1)
        y = jnp.concatenate([ya_r[...], yb_r[...]], axis=1)
        h = _relu(dv_r[...] * (agg + y) + b_r[...])
        yn = dv_r[...] * jnp.dot(h, w_r[...], preferred_element_type=F32)
        if len(outs) == 2:
            outs[0][...], outs[1][...] = _split2(yn)
        else:
            outs[0][...] = yn

    n_outs = 2 if nout == 256 else 1
    return pl.pallas_call(
        body,
        out_shape=[jax.ShapeDtypeStruct((NP_CLL, 128), F32)] * n_outs,
    )(agg_p[0, 0], agg_p[0, 1], agg_p[1, 0], agg_p[1, 1], ya, yb, dinv, b, w)


def _tc_cll_fin(agg_p, y4, dinv, b):
    def body(a0_r, a1_r, y_r, dv_r, b_r, o_r):
        o_r[...] = _relu(dv_r[...] * (a0_r[...] + a1_r[...] + y_r[...]) + b_r[...])

    return pl.pallas_call(
        body, out_shape=jax.ShapeDtypeStruct((NP_CLL, 128), F32),
    )(agg_p[0, 0], agg_p[0, 1], y4, dinv, b)


def _tc_cll_tail(xp, w1, b1, w2, b2, w3, b3):
    nk = xp.shape[1] // CH

    def body(x_r, w1_r, b1_r, w2_r, b2_r, w3_r, b3_r, o_r, accs):
        kk = pl.program_id(0)

        @pl.when(kk == 0)
        def _():
            accs[...] = jnp.zeros_like(accs)

        accs[...] += jnp.dot(x_r[...], w1_r[...], preferred_element_type=F32)

        @pl.when(kk == nk - 1)
        def _():
            t = _relu(accs[...] + b1_r[...])
            t = _relu(jnp.dot(t, w2_r[...], preferred_element_type=F32) + b2_r[...])
            o_r[...] = _relu(jnp.dot(t, w3_r[...], preferred_element_type=F32)
                             + b3_r[...])

    return pl.pallas_call(
        body,
        grid=(nk,),
        in_specs=[
            pl.BlockSpec((8, CH), lambda k: (0, k)),
            pl.BlockSpec((CH, 1000), lambda k: (k, 0)),
            pl.BlockSpec((1, 1000), lambda k: (0, 0)),
            pl.BlockSpec((1000, 1000), lambda k: (0, 0)),
            pl.BlockSpec((1, 1000), lambda k: (0, 0)),
            pl.BlockSpec((1000, 100), lambda k: (0, 0)),
            pl.BlockSpec((1, 100), lambda k: (0, 0)),
        ],
        out_specs=pl.BlockSpec((8, 100), lambda k: (0, 0)),
        out_shape=jax.ShapeDtypeStruct((8, 100), F32),
        scratch_shapes=[pltpu.VMEM((8, 1000), F32)],
    )(xp, w1, b1, w2, b2, w3, b3)


_BM = 1280


def _tc_mol_h1(a0, a1, x, wrel, wroot, b):
    nb = NP_B // _BM

    def body(a0_r, a1_r, x_r, wr_r, wt_r, b_r, oa_r, ob_r):
        h = _relu(jnp.dot(a0_r[...] + a1_r[...], wr_r[...],
                          preferred_element_type=F32)
                  + jnp.dot(x_r[...], wt_r[...], preferred_element_type=F32)
                  + b_r[...])
        i = pl.program_id(0)
        rid = i * _BM + lax.broadcasted_iota(I32, (_BM, 1), 0)
        h = jnp.where(rid < N_MOL, h, 0.0)
        oa_r[...] = h[:, :128]
        ob_r[...] = h[:, 128:256]

    return pl.pallas_call(
        body,
        grid=(nb,),
        in_specs=[
            pl.BlockSpec((_BM, 128), lambda i: (i, 0)),
            pl.BlockSpec((_BM, 128), lambda i: (i, 0)),
            pl.BlockSpec((_BM, 128), lambda i: (i, 0)),
            pl.BlockSpec((128, 256), lambda i: (0, 0)),
            pl.BlockSpec((128, 256), lambda i: (0, 0)),
            pl.BlockSpec((1, 256), lambda i: (0, 0)),
        ],
        out_specs=[pl.BlockSpec((_BM, 128), lambda i: (i, 0)),
                   pl.BlockSpec((_BM, 128), lambda i: (i, 0))],
        out_shape=[jax.ShapeDtypeStruct((NP_B, 128), F32),
                   jax.ShapeDtypeStruct((NP_B, 128), F32)],
    )(a0, a1, x, wrel, wroot, b)


def _tc_mol_h2(a2a0, a2a1, a2b0, a2b1, h1a, h1b, wrel, wroot, b, wlin, blin):
    nb = NP_B // _BM

    def body(aa0_r, aa1_r, ab0_r, ab1_r, ha_r, hb_r, wr_r, wt_r, b_r,
             wl_r, bl_r, o_r, ps):
        i = pl.program_id(0)
        agg = jnp.concatenate([aa0_r[...] + aa1_r[...],
                               ab0_r[...] + ab1_r[...]], axis=1)
        h1 = jnp.concatenate([ha_r[...], hb_r[...]], axis=1)
        h2 = _relu(jnp.dot(agg, wr_r[...], preferred_element_type=F32)
                   + jnp.dot(h1, wt_r[...], preferred_element_type=F32)
                   + b_r[...])
        rid = i * _BM + lax.broadcasted_iota(I32, (_BM, 1), 0)
        h2 = jnp.where(rid < N_MOL, h2, 0.0)

        @pl.when(i == 0)
        def _():
            ps[...] = jnp.zeros_like(ps)

        ps[0:1, :] += jnp.sum(h2, axis=0, keepdims=True)

        @pl.when(i == nb - 1)
        def _():
            pool = ps[0:1, :] * (1.0 / N_MOL)
            xm = _relu(jnp.dot(pool, wl_r[...], preferred_element_type=F32)
                       + bl_r[...])
            o_r[...] = jnp.broadcast_to(xm, (8, 100))

    bs128 = pl.BlockSpec((_BM, 128), lambda i: (i, 0))
    return pl.pallas_call(
        body,
        grid=(nb,),
        in_specs=[
            bs128, bs128, bs128, bs128, bs128, bs128,
            pl.BlockSpec((256, 256), lambda i: (0, 0)),
            pl.BlockSpec((256, 256), lambda i: (0, 0)),
            pl.BlockSpec((1, 256), lambda i: (0, 0)),
            pl.BlockSpec((256, 100), lambda i: (0, 0)),
            pl.BlockSpec((1, 100), lambda i: (0, 0)),
        ],
        out_specs=pl.BlockSpec((8, 100), lambda i: (0, 0)),
        out_shape=jax.ShapeDtypeStruct((8, 100), F32),
        scratch_shapes=[pltpu.VMEM((8, 256), F32)],
    )(a2a0, a2a1, a2b0, a2b1, h1a, h1b, wrel, wroot, b, wlin, blin)


def _tc_bio_prep(x, dinv):
    nb = NP_B // _BM

    def body(x_r, dv_r, o_r):
        o_r[...] = dv_r[...] * x_r[...]

    return pl.pallas_call(
        body,
        grid=(nb,),
        in_specs=[pl.BlockSpec((_BM, 128), lambda i: (i, 0)),
                  pl.BlockSpec((_BM, 1), lambda i: (i, 0))],
        out_specs=pl.BlockSpec((_BM, 128), lambda i: (i, 0)),
        out_shape=jax.ShapeDtypeStruct((NP_B, 128), F32),
    )(x, dinv)


def _tc_bio_zrow(a0, a1, dinvx, dinv, wvec, d99, w1, b1, w2, b2):
    nb = NP_B // _BM

    def body(a0_r, a1_r, dx_r, dv_r, wv_r, d99_r, w1_r, b1_r, w2_r, b2_r,
             o_r, zs):
        i = pl.program_id(0)
        h1 = _relu(dv_r[...] * jnp.dot(a0_r[...] + a1_r[...] + dx_r[...],
                                       w1_r[...], preferred_element_type=F32)
                   + b1_r[...])
        y2 = dv_r[...] * jnp.dot(h1, w2_r[...], preferred_element_type=F32)

        @pl.when(i == 0)
        def _():
            zs[...] = jnp.zeros_like(zs)

        zs[0:1, :] += jnp.dot(wv_r[...], y2, preferred_element_type=F32)

        @pl.when(i == nb - 1)
        def _():
            t = _relu(d99_r[...] * zs[0:1, :] + b2_r[...])
            o_r[...] = jnp.broadcast_to(t, (8, 256))

    return pl.pallas_call(
        body,
        grid=(nb,),
        in_specs=[
            pl.BlockSpec((_BM, 128), lambda i: (i, 0)),
            pl.BlockSpec((_BM, 128), lambda i: (i, 0)),
            pl.BlockSpec((_BM, 128), lambda i: (i, 0)),
            pl.BlockSpec((_BM, 1), lambda i: (i, 0)),
            pl.BlockSpec((1, _BM), lambda i: (0, i)),
            pl.BlockSpec((1, 1), lambda i: (0, 0)),
            pl.BlockSpec((128, 208), lambda i: (0, 0)),
            pl.BlockSpec((1, 208), lambda i: (0, 0)),
            pl.BlockSpec((208, 256), lambda i: (0, 0)),
            pl.BlockSpec((1, 256), lambda i: (0, 0)),
        ],
        out_specs=pl.BlockSpec((8, 256), lambda i: (0, 0)),
        out_shape=jax.ShapeDtypeStruct((8, 256), F32),
        scratch_shapes=[pltpu.VMEM((8, 256), F32)],
    )(a0, a1, dinvx, dinv, wvec, d99, w1, b1, w2, b2)


def _tc_final(xm, tbio, wbl, bbl, xcll,
              wd1, bd1, wd2, bd2, wc1, bc1, wc2, bc2):
    def body(xm_r, t_r, wbl_r, bbl_r, xc_r,
             wd1_r, bd1_r, wd2_r, bd2_r, wc1_r, bc1_r, wc2_r, bc2_r, o_r):
        t = t_r[0:1, :]
        xb = _relu(jnp.dot(t, wbl_r[...], preferred_element_type=F32) + bbl_r[...])
        xd = jnp.concatenate([xm_r[0:1, :], xb], axis=1)
        xd = _relu(jnp.dot(xd, wd1_r[...], preferred_element_type=F32) + bd1_r[...])
        xd = _relu(jnp.dot(xd, wd2_r[...], preferred_element_type=F32) + bd2_r[...])
        xc = jnp.concatenate([xd, xc_r[0:1, :]], axis=1)
        xc = _relu(jnp.dot(xc, wc1_r[...], preferred_element_type=F32) + bc1_r[...])
        o_r[...] = _relu(jnp.dot(xc, wc2_r[...], preferred_element_type=F32)
                         + bc2_r[...])

    return pl.pallas_call(
        body, out_shape=jax.ShapeDtypeStruct((1, 1), F32),
    )(xm, tbio, wbl, bbl, xcll, wd1, bd1, wd2, bd2, wc1, bc1, wc2, bc2)


def _pad_edges(src, dst, ep, n_real, dst_span):
    npad = ep - src.shape[0]
    i = jnp.arange(npad, dtype=I32)
    return (jnp.concatenate([src, n_real + (i % dst_span)]),
            jnp.concatenate([dst, n_real + (i % dst_span)]))


def _padw(w, r, c):
    return jnp.pad(w, ((0, r - w.shape[0]), (0, c - w.shape[1])))


def _padb(b, c):
    return jnp.pad(b, (0, c - b.shape[0]))[None, :]


def kernel(cll_x, mol_x, mol_edge_attr, bio_x, params, cll_edge_index,
           mol_edge_index, mol_batch, bio_edge_index):
    p = params

    xc = jnp.pad(cll_x, ((0, NP_CLL - N_CLL), (0, 0)))
    xm0 = jnp.pad(mol_x, ((0, NP_B - N_MOL), (0, 0)))
    xb0 = jnp.pad(bio_x, ((0, NP_B - N_BIO), (0, 0)))
    csrc, cdst = _pad_edges(cll_edge_index[0], cll_edge_index[1], EP_CLL, N_CLL, 5)
    msrc, mdst = _pad_edges(mol_edge_index[0], mol_edge_index[1], EP_MOL, N_MOL, 128)
    bsrc, bdst = _pad_edges(bio_edge_index[0], bio_edge_index[1], EP_BIO, N_BIO, 128)

    cW = [_padw(p['cll_W1'], 256, 256), _padw(p['cll_W2'], 256, 256),
          _padw(p['cll_W3'], 256, 256), _padw(p['cll_W4'], 256, 128)]
    cb = [_padb(p['cll_b1'], 256), _padb(p['cll_b2'], 256),
          _padb(p['cll_b3'], 256), _padb(p['cll_b4'], 128)]

    degc_p, degb_p, flg_p = _sc_degrees(cdst, bsrc, bdst)
    degc_p = degc_p.reshape(NC, NP_CLL)
    degb_p = degb_p.reshape(NC, NP_B)
    flg_sum = flg_p[:NP_B] + flg_p[NP_B:]
    agg1_p = _sc_segsum((xm0,), msrc, mdst, n_out=NP_B, ep=EP_MOL)[0]

    dinvc = lax.rsqrt(degc_p[0] + degc_p[1] + 1.0)[:, None]
    dinvb = lax.rsqrt(degb_p[0] + degb_p[1] + 1.0)[:, None]

    ya, yb = _tc_y1c(xc, cW[0], dinvc)
    for li in range(3):
        a_p = _sc_segsum((ya, yb), csrc, cdst, n_out=NP_CLL, ep=EP_CLL)
        nxt = _tc_cll_step(a_p, ya, yb, dinvc, cb[li], cW[li + 1])
        if li < 2:
            ya, yb = nxt
        else:
            y4 = nxt[0]
    a_p = _sc_segsum((y4,), csrc, cdst, n_out=NP_CLL, ep=EP_CLL)
    h4 = _tc_cll_fin(a_p, y4, dinvc, cb[3])
    xflat = h4[:N_CLL, :3].reshape(1, -1)
    xp = jnp.pad(xflat, ((0, 7), (0, 15)))
    w1p = jnp.pad(p['cll_lin1_W'], ((0, 15), (0, 0)))
    x_cll = _tc_cll_tail(xp, w1p, p['cll_lin1_b'][None], p['cll_lin2_W'],
                         p['cll_lin2_b'][None], p['cll_lin3_W'],
                         p['cll_lin3_b'][None])

    h1a, h1b = _tc_mol_h1(agg1_p[0], agg1_p[1], xm0,
                          _padw(p['mol_Wrel1'], 128, 256),
                          _padw(p['mol_Wroot1'], 128, 256),
                          _padb(p['mol_b1'], 256))
    a2_p = _sc_segsum((h1a, h1b), msrc, mdst, n_out=NP_B, ep=EP_MOL)
    a2a_p, a2b_p = a2_p[0], a2_p[1]
    xm = _tc_mol_h2(a2a_p[0], a2a_p[1], a2b_p[0], a2b_p[1], h1a, h1b,
                    _padw(p['mol_Wrel2'], 256, 256),
                    _padw(p['mol_Wroot2'], 256, 256),
                    _padb(p['mol_b2'], 256),
                    _padw(p['mol_lin_W'], 256, 100),
                    p['mol_lin_b'][None])

    dinvx = _tc_bio_prep(xb0, dinvb)
    accb_p = _sc_segsum((dinvx,), bsrc, bdst, n_out=NP_B, ep=EP_BIO)[0]
    wvec = (flg_sum + (jnp.arange(NP_B) == LAST).astype(F32))[None, :]
    tbio = _tc_bio_zrow(accb_p[0], accb_p[1], dinvx, dinvb, wvec,
                        dinvb[LAST:LAST + 1],
                        _padw(p['bio_W1'], 128, 208), _padb(p['bio_b1'], 208),
                        _padw(p['bio_W2'], 208, 256), _padb(p['bio_b2'], 256))

    out = _tc_final(xm, tbio, _padw(p['bio_lin_W'], 256, 100),
                    p['bio_lin_b'][None], x_cll,
                    p['drug1_W'], p['drug1_b'][None],
                    p['drug2_W'], p['drug2_b'][None],
                    p['cat1_W'], p['cat1_b'][None],
                    p['cat2_W'], p['cat2_b'][None])
    return out


# speedup vs baseline: 26.5748x; 2.1867x over previous
import functools

import jax
import jax.numpy as jnp
from jax import lax
from jax.experimental import pallas as pl
from jax.experimental.pallas import tpu as pltpu
from jax.experimental.pallas import tpu_sc as plsc

F32 = jnp.float32
I32 = jnp.int32

NC, NS, LN = 2, 16, 16
NW = NC * NS
CH = 128

N_CLL, E_CLL = 3451, 55216
N_MOL, E_MOL = 10000, 320000
N_BIO, E_BIO = 10000, 640000
NP_CLL = 4096
NP_B = 10240
EP_CLL = 65536
EP_MOL = 327680
EP_BIO = 655360
LAST = N_BIO - 1

_mesh = plsc.VectorSubcoreMesh(core_axis_name="c", subcore_axis_name="s")


def _relu(x):
    return jnp.maximum(x, 0.0)


def _zero_vmem2d(ref, rows, d):
    zv = jnp.zeros((LN,), F32)

    def body(i, _):
        for cc in range(d // LN):
            ref[i, pl.ds(cc * LN, LN)] = zv
        return 0

    lax.fori_loop(0, rows, body, 0)


_SUP = 16


@functools.partial(jax.jit, static_argnames=("n_out", "eps"))
def _sc_segsum(phases, *, n_out, eps):
    d = 128
    nt = len(eps)
    rows_ps = n_out // NS

    @functools.partial(
        pl.kernel,
        out_type=tuple(jax.ShapeDtypeStruct((NC, n_out, d), F32)
                       for _ in range(nt)),
        mesh=_mesh,
        scratch_types=[
            pltpu.VMEM((_SUP, CH), I32),
            pltpu.VMEM((_SUP, CH), I32),
            pltpu.VMEM((CH, d), F32),
            pltpu.VMEM((CH, d), F32),
            pltpu.VMEM((64, d), F32),
            pltpu.VMEM_SHARED((n_out, d), F32),
            pltpu.SemaphoreType.DMA,
            pltpu.SemaphoreType.DMA,
        ],
    )
    def k(*refs):
        ins = refs[:3 * nt]
        (sidx, didx, rows0, rows1, zbuf, acc, sem0, sem1) = refs[3 * nt + nt:]
        out_hs = refs[3 * nt:3 * nt + nt]
        c = lax.axis_index("c")
        s = lax.axis_index("s")
        _zero_vmem2d(zbuf, 64, d)
        for t in range(nt):
            table_h, src_h, dst_h = ins[3 * t:3 * t + 3]
            npt = eps[t] // NW
            nch = npt // CH
            sup_sz = min(_SUP, nch)
            nsup = nch // sup_sz
            sbase = (c * NS + s) * nch
            for q in range(rows_ps // 64):
                pltpu.sync_copy(zbuf, acc.at[pl.ds(s * rows_ps + q * 64, 64)])
            plsc.subcore_barrier()

            def sup(q, _, t=t, table_h=table_h, src_h=src_h, dst_h=dst_h,
                    sup_sz=sup_sz, sbase=sbase):
                r0 = sbase + q * sup_sz
                pltpu.sync_copy(src_h.at[pl.ds(r0, sup_sz)],
                                sidx.at[pl.ds(0, sup_sz)])
                pltpu.sync_copy(dst_h.at[pl.ds(r0, sup_sz)],
                                didx.at[pl.ds(0, sup_sz)])
                bufs = (rows0, rows1)
                sems = (sem0, sem1)
                h = pltpu.async_copy(table_h.at[sidx.at[0]], bufs[0],
                                     sems[0])
                for jj in range(sup_sz):
                    if jj + 1 < sup_sz:
                        hn = pltpu.async_copy(
                            table_h.at[sidx.at[jj + 1]],
                            bufs[(jj + 1) % 2], sems[(jj + 1) % 2])
                    h.wait()
                    pltpu.sync_copy(bufs[jj % 2], acc.at[didx.at[jj]],
                                    add=True)
                    if jj + 1 < sup_sz:
                        h = hn
                return 0

            lax.fori_loop(0, nsup, sup, 0)
            plsc.subcore_barrier()
            pltpu.sync_copy(acc.at[pl.ds(s * rows_ps, rows_ps)],
                            out_hs[t].at[c, pl.ds(s * rows_ps, rows_ps)])
            plsc.subcore_barrier()

    args = []
    for (table, src2, dst2) in phases:
        args += [table, src2, dst2]
    return k(*args)


@jax.jit
def _sc_degrees(cdst, bsrc, bdst):
    nch_c = EP_CLL // NW // CH
    nch_b = EP_BIO // NW // CH
    rc = NP_CLL // NS
    rb = NP_B // NS

    @functools.partial(
        pl.kernel,
        out_type=(jax.ShapeDtypeStruct((NC * NP_CLL,), F32),
                  jax.ShapeDtypeStruct((NC * NP_B,), F32),
                  jax.ShapeDtypeStruct((NC * NP_B,), F32)),
        mesh=_mesh,
        scratch_types=[
            pltpu.VMEM((_SUP, CH), I32),
            pltpu.VMEM((_SUP, CH), I32),
            pltpu.VMEM((CH,), F32),
            pltpu.VMEM((_SUP, CH), F32),
            pltpu.VMEM((rb,), F32),
            pltpu.VMEM_SHARED((NP_CLL,), F32),
            pltpu.VMEM_SHARED((NP_B,), F32),
            pltpu.VMEM_SHARED((NP_B,), F32),
        ],
    )
    def k(cdst_h, bsrc_h, bdst_h, degc_h, degb_h, flg_h,
          sidx, didx, ones, val, zb, accc, accb, accf):
        c = lax.axis_index("c")
        s = lax.axis_index("s")
        for kk in range(CH // LN):
            ones[pl.ds(kk * LN, LN)] = jnp.ones((LN,), F32)

        def zbody(i, _):
            zb[pl.ds(i * LN, LN)] = jnp.zeros((LN,), F32)
            return 0

        lax.fori_loop(0, rb // LN, zbody, 0)
        pltpu.sync_copy(zb.at[pl.ds(0, rc)], accc.at[pl.ds(s * rc, rc)])
        pltpu.sync_copy(zb, accb.at[pl.ds(s * rb, rb)])
        pltpu.sync_copy(zb, accf.at[pl.ds(s * rb, rb)])
        plsc.subcore_barrier()

        cbase = (c * NS + s) * (nch_c // _SUP)

        def supc(q, _):
            pltpu.sync_copy(cdst_h.at[pl.ds((cbase + q) * _SUP, _SUP)], didx)
            for jj in range(_SUP):
                pltpu.sync_copy(ones, accc.at[didx.at[jj]], add=True)
            return 0

        lax.fori_loop(0, nch_c // _SUP, supc, 0)

        bbase = (c * NS + s) * (nch_b // _SUP)

        def supb(q, _):
            r0 = (bbase + q) * _SUP
            pltpu.sync_copy(bsrc_h.at[pl.ds(r0, _SUP)], sidx)
            pltpu.sync_copy(bdst_h.at[pl.ds(r0, _SUP)], didx)
            for jj in range(_SUP):
                pltpu.sync_copy(ones, accb.at[didx.at[jj]], add=True)
                for kk in range(CH // LN):
                    dv = didx[jj, pl.ds(kk * LN, LN)]
                    val[jj, pl.ds(kk * LN, LN)] = jnp.where(
                        dv == LAST, jnp.ones((LN,), F32), jnp.zeros((LN,), F32))
                pltpu.sync_copy(val.at[jj], accf.at[sidx.at[jj]], add=True)
            return 0

        lax.fori_loop(0, nch_b // _SUP, supb, 0)
        plsc.subcore_barrier()
        pltpu.sync_copy(accc.at[pl.ds(s * rc, rc)],
                        degc_h.at[pl.ds(c * NP_CLL + s * rc, rc)])
        pltpu.sync_copy(accb.at[pl.ds(s * rb, rb)],
                        degb_h.at[pl.ds(c * NP_B + s * rb, rb)])
        pltpu.sync_copy(accf.at[pl.ds(s * rb, rb)],
                        flg_h.at[pl.ds(c * NP_B + s * rb, rb)])

    return k(cdst, bsrc, bdst)


def _split2(h):
    return h[:, :128], h[:, 128:256]


def _tc_y1c(xc, w, dinv):
    def body(x_r, w_r, dv_r, oa_r, ob_r):
        y = dv_r[...] * jnp.dot(x_r[...], w_r[...], preferred_element_type=F32)
        oa_r[...], ob_r[...] = _split2(y)

    return pl.pallas_call(
        body, out_shape=[jax.ShapeDtypeStruct((NP_CLL, 128), F32),
                         jax.ShapeDtypeStruct((NP_CLL, 128), F32)],
    )(xc, w, dinv)


def _tc_cll_step(agg_p, ya, yb, dinv, b, w):
    nout = w.shape[1]

    def body(aa0_r, aa1_r, ab0_r, ab1_r, ya_r, yb_r, dv_r, b_r, w_r, *outs):
        agg = jnp.concatenate([aa0_r[...] + aa1_r[...],
                               ab0_r[...] + ab1_r[...]], axis=1)
        y = jnp.concatenate([ya_r[...], yb_r[...]], axis=1)
        h = _relu(dv_r[...] * (agg + y) + b_r[...])
        yn = dv_r[...] * jnp.dot(h, w_r[...], preferred_element_type=F32)
        if len(outs) == 2:
            outs[0][...], outs[1][...] = _split2(yn)
        else:
            outs[0][...] = yn

    n_outs = 2 if nout == 256 else 1
    return pl.pallas_call(
        body,
        out_shape=[jax.ShapeDtypeStruct((NP_CLL, 128), F32)] * n_outs,
    )(agg_p[0][0], agg_p[0][1], agg_p[1][0], agg_p[1][1], ya, yb, dinv, b, w)


def _tc_cll_fin(agg_p, y4, dinv, b):
    def body(a0_r, a1_r, y_r, dv_r, b_r, o_r):
        o_r[...] = _relu(dv_r[...] * (a0_r[...] + a1_r[...] + y_r[...]) + b_r[...])

    return pl.pallas_call(
        body, out_shape=jax.ShapeDtypeStruct((NP_CLL, 128), F32),
    )(agg_p[0][0], agg_p[0][1], y4, dinv, b)


def _tc_cll_tail(xp, w1, xtail, wtail, b1, w2, b2, w3, b3):
    nk = 80

    def body(x_r, w1_r, xt_r, wt_r, b1_r, w2_r, b2_r, w3_r, b3_r, o_r, accs):
        kk = pl.program_id(0)

        @pl.when(kk == 0)
        def _():
            accs[...] = jnp.zeros_like(accs)

        accs[...] += jnp.dot(x_r[...], w1_r[...], preferred_element_type=F32)

        @pl.when(kk == nk - 1)
        def _():
            t = accs[...] + jnp.dot(xt_r[...], wt_r[...],
                                    preferred_element_type=F32)
            t = _relu(t + b1_r[...])
            t = _relu(jnp.dot(t, w2_r[...], preferred_element_type=F32) + b2_r[...])
            o_r[...] = _relu(jnp.dot(t, w3_r[...], preferred_element_type=F32)
                             + b3_r[...])

    return pl.pallas_call(
        body,
        grid=(nk,),
        in_specs=[
            pl.BlockSpec((8, CH), lambda k: (0, k)),
            pl.BlockSpec((CH, 1000), lambda k: (k, 0)),
            pl.BlockSpec((8, CH), lambda k: (0, 0)),
            pl.BlockSpec((CH, 1000), lambda k: (0, 0)),
            pl.BlockSpec((1, 1000), lambda k: (0, 0)),
            pl.BlockSpec((1000, 1000), lambda k: (0, 0)),
            pl.BlockSpec((1, 1000), lambda k: (0, 0)),
            pl.BlockSpec((1000, 100), lambda k: (0, 0)),
            pl.BlockSpec((1, 100), lambda k: (0, 0)),
        ],
        out_specs=pl.BlockSpec((8, 100), lambda k: (0, 0)),
        out_shape=jax.ShapeDtypeStruct((8, 100), F32),
        scratch_shapes=[pltpu.VMEM((8, 1000), F32)],
    )(xp, w1, xtail, wtail, b1, w2, b2, w3, b3)


_BM = 1280


def _tc_mol_h1(a0, a1, x, wrel, wroot, b):
    nb = NP_B // _BM

    def body(a0_r, a1_r, x_r, wr_r, wt_r, b_r, oa_r, ob_r):
        h = _relu(jnp.dot(a0_r[...] + a1_r[...], wr_r[...],
                          preferred_element_type=F32)
                  + jnp.dot(x_r[...], wt_r[...], preferred_element_type=F32)
                  + b_r[...])
        i = pl.program_id(0)
        rid = i * _BM + lax.broadcasted_iota(I32, (_BM, 1), 0)
        h = jnp.where(rid < N_MOL, h, 0.0)
        oa_r[...] = h[:, :128]
        ob_r[...] = h[:, 128:256]

    return pl.pallas_call(
        body,
        grid=(nb,),
        in_specs=[
            pl.BlockSpec((_BM, 128), lambda i: (i, 0)),
            pl.BlockSpec((_BM, 128), lambda i: (i, 0)),
            pl.BlockSpec((_BM, 128), lambda i: (i, 0)),
            pl.BlockSpec((128, 256), lambda i: (0, 0)),
            pl.BlockSpec((128, 256), lambda i: (0, 0)),
            pl.BlockSpec((1, 256), lambda i: (0, 0)),
        ],
        out_specs=[pl.BlockSpec((_BM, 128), lambda i: (i, 0)),
                   pl.BlockSpec((_BM, 128), lambda i: (i, 0))],
        out_shape=[jax.ShapeDtypeStruct((NP_B, 128), F32),
                   jax.ShapeDtypeStruct((NP_B, 128), F32)],
    )(a0, a1, x, wrel, wroot, b)


def _tc_mol_h2(a2a0, a2a1, a2b0, a2b1, h1a, h1b, wrel, wroot, b, wlin, blin):
    nb = NP_B // _BM

    def body(aa0_r, aa1_r, ab0_r, ab1_r, ha_r, hb_r, wr_r, wt_r, b_r,
             wl_r, bl_r, o_r, ps):
        i = pl.program_id(0)
        agg = jnp.concatenate([aa0_r[...] + aa1_r[...],
                               ab0_r[...] + ab1_r[...]], axis=1)
        h1 = jnp.concatenate([ha_r[...], hb_r[...]], axis=1)
        h2 = _relu(jnp.dot(agg, wr_r[...], preferred_element_type=F32)
                   + jnp.dot(h1, wt_r[...], preferred_element_type=F32)
                   + b_r[...])
        rid = i * _BM + lax.broadcasted_iota(I32, (_BM, 1), 0)
        h2 = jnp.where(rid < N_MOL, h2, 0.0)

        @pl.when(i == 0)
        def _():
            ps[...] = jnp.zeros_like(ps)

        ps[0:1, :] += jnp.sum(h2, axis=0, keepdims=True)

        @pl.when(i == nb - 1)
        def _():
            pool = ps[0:1, :] * (1.0 / N_MOL)
            xm = _relu(jnp.dot(pool, wl_r[...], preferred_element_type=F32)
                       + bl_r[...])
            o_r[...] = jnp.broadcast_to(xm, (8, 100))

    bs128 = pl.BlockSpec((_BM, 128), lambda i: (i, 0))
    return pl.pallas_call(
        body,
        grid=(nb,),
        in_specs=[
            bs128, bs128, bs128, bs128, bs128, bs128,
            pl.BlockSpec((256, 256), lambda i: (0, 0)),
            pl.BlockSpec((256, 256), lambda i: (0, 0)),
            pl.BlockSpec((1, 256), lambda i: (0, 0)),
            pl.BlockSpec((256, 100), lambda i: (0, 0)),
            pl.BlockSpec((1, 100), lambda i: (0, 0)),
        ],
        out_specs=pl.BlockSpec((8, 100), lambda i: (0, 0)),
        out_shape=jax.ShapeDtypeStruct((8, 100), F32),
        scratch_shapes=[pltpu.VMEM((8, 256), F32)],
    )(a2a0, a2a1, a2b0, a2b1, h1a, h1b, wrel, wroot, b, wlin, blin)


def _tc_bio_prep(x, dinv):
    nb = NP_B // _BM

    def body(x_r, dv_r, o_r):
        o_r[...] = dv_r[...] * x_r[...]

    return pl.pallas_call(
        body,
        grid=(nb,),
        in_specs=[pl.BlockSpec((_BM, 128), lambda i: (i, 0)),
                  pl.BlockSpec((_BM, 1), lambda i: (i, 0))],
        out_specs=pl.BlockSpec((_BM, 128), lambda i: (i, 0)),
        out_shape=jax.ShapeDtypeStruct((NP_B, 128), F32),
    )(x, dinv)


def _tc_bio_zrow(a0, a1, dinvx, dinv, wvec, d99, w1, b1, w2, b2):
    nb = NP_B // _BM

    def body(a0_r, a1_r, dx_r, dv_r, wv_r, d99_r, w1_r, b1_r, w2_r, b2_r,
             o_r, zs):
        i = pl.program_id(0)
        h1 = _relu(dv_r[...] * jnp.dot(a0_r[...] + a1_r[...] + dx_r[...],
                                       w1_r[...], preferred_element_type=F32)
                   + b1_r[...])
        y2 = dv_r[...] * jnp.dot(h1, w2_r[...], preferred_element_type=F32)

        @pl.when(i == 0)
        def _():
            zs[...] = jnp.zeros_like(zs)

        zs[0:1, :] += jnp.dot(wv_r[...], y2, preferred_element_type=F32)

        @pl.when(i == nb - 1)
        def _():
            t = _relu(d99_r[...] * zs[0:1, :] + b2_r[...])
            o_r[...] = jnp.broadcast_to(t, (8, 256))

    return pl.pallas_call(
        body,
        grid=(nb,),
        in_specs=[
            pl.BlockSpec((_BM, 128), lambda i: (i, 0)),
            pl.BlockSpec((_BM, 128), lambda i: (i, 0)),
            pl.BlockSpec((_BM, 128), lambda i: (i, 0)),
            pl.BlockSpec((_BM, 1), lambda i: (i, 0)),
            pl.BlockSpec((1, _BM), lambda i: (0, i)),
            pl.BlockSpec((1, 1), lambda i: (0, 0)),
            pl.BlockSpec((128, 208), lambda i: (0, 0)),
            pl.BlockSpec((1, 208), lambda i: (0, 0)),
            pl.BlockSpec((208, 256), lambda i: (0, 0)),
            pl.BlockSpec((1, 256), lambda i: (0, 0)),
        ],
        out_specs=pl.BlockSpec((8, 256), lambda i: (0, 0)),
        out_shape=jax.ShapeDtypeStruct((8, 256), F32),
        scratch_shapes=[pltpu.VMEM((8, 256), F32)],
    )(a0, a1, dinvx, dinv, wvec, d99, w1, b1, w2, b2)


def _tc_final(xm, tbio, wbl, bbl, xcll,
              wd1, bd1, wd2, bd2, wc1, bc1, wc2, bc2):
    def body(xm_r, t_r, wbl_r, bbl_r, xc_r,
             wd1_r, bd1_r, wd2_r, bd2_r, wc1_r, bc1_r, wc2_r, bc2_r, o_r):
        t = t_r[0:1, :]
        xb = _relu(jnp.dot(t, wbl_r[...], preferred_element_type=F32) + bbl_r[...])
        xd = jnp.concatenate([xm_r[0:1, :], xb], axis=1)
        xd = _relu(jnp.dot(xd, wd1_r[...], preferred_element_type=F32) + bd1_r[...])
        xd = _relu(jnp.dot(xd, wd2_r[...], preferred_element_type=F32) + bd2_r[...])
        xc = jnp.concatenate([xd, xc_r[0:1, :]], axis=1)
        xc = _relu(jnp.dot(xc, wc1_r[...], preferred_element_type=F32) + bc1_r[...])
        o_r[...] = _relu(jnp.dot(xc, wc2_r[...], preferred_element_type=F32)
                         + bc2_r[...])

    return pl.pallas_call(
        body, out_shape=jax.ShapeDtypeStruct((1, 1), F32),
    )(xm, tbio, wbl, bbl, xcll, wd1, bd1, wd2, bd2, wc1, bc1, wc2, bc2)


def _pad_edges(src, dst, ep, n_real, dst_span):
    npad = ep - src.shape[0]
    i = jnp.arange(npad, dtype=I32)
    return (jnp.concatenate([src, n_real + (i % dst_span)]),
            jnp.concatenate([dst, n_real + (i % dst_span)]))


def _padw(w, r, c):
    return jnp.pad(w, ((0, r - w.shape[0]), (0, c - w.shape[1])))


def _padb(b, c):
    return jnp.pad(b, (0, c - b.shape[0]))[None, :]


def kernel(cll_x, mol_x, mol_edge_attr, bio_x, params, cll_edge_index,
           mol_edge_index, mol_batch, bio_edge_index):
    p = params

    xc = jnp.pad(cll_x, ((0, NP_CLL - N_CLL), (0, 0)))
    xm0 = jnp.pad(mol_x, ((0, NP_B - N_MOL), (0, 0)))
    xb0 = jnp.pad(bio_x, ((0, NP_B - N_BIO), (0, 0)))
    csrc, cdst = _pad_edges(cll_edge_index[0], cll_edge_index[1], EP_CLL, N_CLL, 512)
    msrc, mdst = _pad_edges(mol_edge_index[0], mol_edge_index[1], EP_MOL, N_MOL, 128)
    bsrc, bdst = _pad_edges(bio_edge_index[0], bio_edge_index[1], EP_BIO, N_BIO, 128)
    csrc2, cdst2 = csrc.reshape(-1, CH), cdst.reshape(-1, CH)
    msrc2, mdst2 = msrc.reshape(-1, CH), mdst.reshape(-1, CH)
    bsrc2, bdst2 = bsrc.reshape(-1, CH), bdst.reshape(-1, CH)

    cW = [_padw(p['cll_W1'], 256, 256), _padw(p['cll_W2'], 256, 256),
          _padw(p['cll_W3'], 256, 256), _padw(p['cll_W4'], 256, 128)]
    cb = [_padb(p['cll_b1'], 256), _padb(p['cll_b2'], 256),
          _padb(p['cll_b3'], 256), _padb(p['cll_b4'], 128)]

    degc_p, degb_p, flg_p = _sc_degrees(cdst2, bsrc2, bdst2)
    degc_p = degc_p.reshape(NC, NP_CLL)
    degb_p = degb_p.reshape(NC, NP_B)
    flg_sum = flg_p[:NP_B] + flg_p[NP_B:]
    agg1_p = _sc_segsum(((xm0, msrc2, mdst2),), n_out=NP_B, eps=(EP_MOL,))[0]

    dinvc = lax.rsqrt(degc_p[0] + degc_p[1] + 1.0)[:, None]
    dinvb = lax.rsqrt(degb_p[0] + degb_p[1] + 1.0)[:, None]

    ya, yb = _tc_y1c(xc, cW[0], dinvc)
    for li in range(3):
        a_p = _sc_segsum(((ya, csrc2, cdst2), (yb, csrc2, cdst2)),
                         n_out=NP_CLL, eps=(EP_CLL, EP_CLL))
        nxt = _tc_cll_step(a_p, ya, yb, dinvc, cb[li], cW[li + 1])
        if li < 2:
            ya, yb = nxt
        else:
            y4 = nxt[0]
    a_p = _sc_segsum(((y4, csrc2, cdst2),), n_out=NP_CLL, eps=(EP_CLL,))
    h4 = _tc_cll_fin(a_p, y4, dinvc, cb[3])
    xflat = h4[:N_CLL, :3].reshape(1, -1)
    xp = jnp.pad(xflat, ((0, 7), (0, 15)))
    xtail = xp[:, 10240:10368]
    wtail = jnp.pad(p['cll_lin1_W'][10240:, :], ((0, 15), (0, 0)))
    x_cll = _tc_cll_tail(xp, p['cll_lin1_W'], xtail, wtail,
                         p['cll_lin1_b'][None], p['cll_lin2_W'],
                         p['cll_lin2_b'][None], p['cll_lin3_W'],
                         p['cll_lin3_b'][None])

    h1a, h1b = _tc_mol_h1(agg1_p[0], agg1_p[1], xm0,
                          _padw(p['mol_Wrel1'], 128, 256),
                          _padw(p['mol_Wroot1'], 128, 256),
                          _padb(p['mol_b1'], 256))
    dinvx = _tc_bio_prep(xb0, dinvb)
    a2a_p, a2b_p, accb_p = _sc_segsum(
        ((h1a, msrc2, mdst2), (h1b, msrc2, mdst2), (dinvx, bsrc2, bdst2)),
        n_out=NP_B, eps=(EP_MOL, EP_MOL, EP_BIO))
    xm = _tc_mol_h2(a2a_p[0], a2a_p[1], a2b_p[0], a2b_p[1], h1a, h1b,
                    _padw(p['mol_Wrel2'], 256, 256),
                    _padw(p['mol_Wroot2'], 256, 256),
                    _padb(p['mol_b2'], 256),
                    _padw(p['mol_lin_W'], 256, 100),
                    p['mol_lin_b'][None])

    wvec = (flg_sum + (jnp.arange(NP_B) == LAST).astype(F32))[None, :]
    tbio = _tc_bio_zrow(accb_p[0], accb_p[1], dinvx, dinvb, wvec,
                        dinvb[LAST:LAST + 1],
                        _padw(p['bio_W1'], 128, 208), _padb(p['bio_b1'], 208),
                        _padw(p['bio_W2'], 208, 256), _padb(p['bio_b2'], 256))

    out = _tc_final(xm, tbio, _padw(p['bio_lin_W'], 256, 100),
                    p['bio_lin_b'][None], x_cll,
                    p['drug1_W'], p['drug1_b'][None],
                    p['drug2_W'], p['drug2_b'][None],
                    p['cat1_W'], p['cat1_b'][None],
                    p['cat2_W'], p['cat2_b'][None])
    return out
